# baseline (speedup 1.0000x reference)
import functools

import jax
import jax.numpy as jnp
from jax import lax
from jax.experimental import pallas as pl
from jax.experimental.pallas import tpu as pltpu

_F32_MAX_BITS = 0x7F7FFFFF


def _mm(a, b):
    return lax.dot_general(
        a, b, (((1,), (0,)), ((), ())),
        precision=lax.Precision.DEFAULT,
        preferred_element_type=jnp.float32,
    )


def _gram(a, b):
    return lax.dot_general(
        a, b, (((1,), (1,)), ((), ())),
        precision=lax.Precision.DEFAULT,
        preferred_element_type=jnp.float32,
    )


def _topk_mask(S, k, lo_ref, hi_ref, clo_ref, chi_ref):
    assert k <= 64
    n = S.shape[1]
    g = S
    while g.shape[1] > 64:
        half = g.shape[1] // 2
        g = jnp.maximum(g[:, :half], g[:, half:])
    lo0 = lax.bitcast_convert_type(jnp.min(g, axis=1, keepdims=True), jnp.int32)
    hi0 = lax.bitcast_convert_type(jnp.max(g, axis=1, keepdims=True), jnp.int32)
    lo_ref[...] = lo0
    hi_ref[...] = hi0
    clo_ref[...] = jnp.full(lo0.shape, float(n), jnp.float32)
    chi_ref[...] = jnp.zeros(lo0.shape, jnp.float32)

    def probe(lo, hi, clo, chi, mid):
        t = lax.bitcast_convert_type(mid, jnp.float32)
        cnt = jnp.sum((S >= t).astype(jnp.float32), axis=1, keepdims=True)
        ge = cnt >= float(k)
        eq = cnt == float(k)
        new_lo = jnp.where(ge, mid, lo)
        new_hi = jnp.where(eq, mid, jnp.where(ge, hi, mid - 1))
        new_clo = jnp.where(ge, cnt, clo)
        new_chi = jnp.where(ge, chi, cnt)
        return new_lo, new_hi, new_clo, new_chi

    def body(_):
        lo, hi = lo_ref[...], hi_ref[...]
        clo, chi = clo_ref[...], chi_ref[...]
        mid = lo + (hi - lo + 1) // 2
        lo, hi, clo, chi = probe(lo, hi, clo, chi, mid)
        span = (hi - lo + 1).astype(jnp.float32)
        off = (clo - float(k)) / (clo - chi) * span
        offi = jnp.clip(off.astype(jnp.int32), 1, jnp.maximum(hi - lo, 1))
        lo, hi, clo, chi = probe(lo, hi, clo, chi, lo + offi)
        span = (hi - lo + 1).astype(jnp.float32)
        off = (clo - float(k)) / (clo - chi) * span
        offi = jnp.clip(off.astype(jnp.int32), 1, jnp.maximum(hi - lo, 1))
        lo, hi, clo, chi = probe(lo, hi, clo, chi, lo + offi)
        lo_ref[...], hi_ref[...] = lo, hi
        clo_ref[...], chi_ref[...] = clo, chi
        return jnp.max(hi - lo) > 0

    lax.while_loop(lambda c: c, body, jnp.max(hi0 - lo0) > 0)
    t = lax.bitcast_convert_type(lo_ref[...], jnp.float32)
    return jnp.where(S >= t, S, 0.0)


def _embed_body(x_ref, w1_ref, b1_ref, w2_ref, b2_ref, h_ref):
    h = jax.nn.relu(_mm(x_ref[0], w1_ref[...]) + b1_ref[...])
    h_ref[0] = jax.nn.relu(_mm(h, w2_ref[...]) + b2_ref[...])


def _msgpass_body(hrows_ref, hfull_ref, wu_ref, wm_ref, out_ref,
                  lo_ref, hi_ref, clo_ref, chi_ref, *, k):
    hr = hrows_ref[0]
    hb = hfull_ref[0]
    A = _topk_mask(_gram(hr, hb), k, lo_ref, hi_ref, clo_ref, chi_ref)
    m = _mm(A, hb)
    out_ref[0] = jax.nn.relu(_mm(hr, wu_ref[...]) + _mm(m, wm_ref[...]))


def _final_body(hrows_ref, hfull_ref, out_ref,
                lo_ref, hi_ref, clo_ref, chi_ref, *, k):
    out_ref[0] = _topk_mask(_gram(hrows_ref[0], hfull_ref[0]), k,
                            lo_ref, hi_ref, clo_ref, chi_ref)


def kernel(x, W_emb1, b_emb1, W_emb2, b_emb2, W_msg, W_upd):
    bs, n, f = x.shape
    hid = W_emb1.shape[1]
    iters = W_msg.shape[0]
    k = 50
    r = min(1024, n)
    rb = n // r

    b1 = b_emb1.reshape(1, hid)
    b2 = b_emb2.reshape(1, hid)

    h = pl.pallas_call(
        _embed_body,
        grid=(bs,),
        in_specs=[
            pl.BlockSpec((1, n, f), lambda b: (b, 0, 0)),
            pl.BlockSpec((f, hid), lambda b: (0, 0)),
            pl.BlockSpec((1, hid), lambda b: (0, 0)),
            pl.BlockSpec((hid, hid), lambda b: (0, 0)),
            pl.BlockSpec((1, hid), lambda b: (0, 0)),
        ],
        out_specs=pl.BlockSpec((1, n, hid), lambda b: (b, 0, 0)),
        out_shape=jax.ShapeDtypeStruct((bs, n, hid), jnp.float32),
    )(x, W_emb1, b1, W_emb2, b2)

    for i in range(iters):
        h = pl.pallas_call(
            functools.partial(_msgpass_body, k=k),
            grid=(bs, rb),
            in_specs=[
                pl.BlockSpec((1, r, hid), lambda b, j: (b, j, 0)),
                pl.BlockSpec((1, n, hid), lambda b, j: (b, 0, 0)),
                pl.BlockSpec((hid, hid), lambda b, j: (0, 0)),
                pl.BlockSpec((hid, hid), lambda b, j: (0, 0)),
            ],
            out_specs=pl.BlockSpec((1, r, hid), lambda b, j: (b, j, 0)),
            out_shape=jax.ShapeDtypeStruct((bs, n, hid), jnp.float32),
            scratch_shapes=[pltpu.VMEM((r, 1), jnp.int32),
                            pltpu.VMEM((r, 1), jnp.int32),
                            pltpu.VMEM((r, 1), jnp.float32),
                            pltpu.VMEM((r, 1), jnp.float32)],
        )(h, h, W_upd[i], W_msg[i])

    return pl.pallas_call(
        functools.partial(_final_body, k=k),
        grid=(bs, rb),
        in_specs=[
            pl.BlockSpec((1, r, hid), lambda b, j: (b, j, 0)),
            pl.BlockSpec((1, n, hid), lambda b, j: (b, 0, 0)),
        ],
        out_specs=pl.BlockSpec((1, r, n), lambda b, j: (b, j, 0)),
        out_shape=jax.ShapeDtypeStruct((bs, n, n), jnp.float32),
        scratch_shapes=[pltpu.VMEM((r, 1), jnp.int32),
                        pltpu.VMEM((r, 1), jnp.int32),
                        pltpu.VMEM((r, 1), jnp.float32),
                        pltpu.VMEM((r, 1), jnp.float32)],
    )(h, h)


# speedup vs baseline: 1.0378x; 1.0378x over previous
import functools

import jax
import jax.numpy as jnp
from jax import lax
from jax.experimental import pallas as pl
from jax.experimental.pallas import tpu as pltpu

_F32_MAX_BITS = 0x7F7FFFFF


def _mm(a, b):
    return lax.dot_general(
        a, b, (((1,), (0,)), ((), ())),
        precision=lax.Precision.DEFAULT,
        preferred_element_type=jnp.float32,
    )


def _gram(a, b):
    return lax.dot_general(
        a, b, (((1,), (1,)), ((), ())),
        precision=lax.Precision.DEFAULT,
        preferred_element_type=jnp.float32,
    )


def _topk_mask(S, k, lo_ref, hi_ref, clo_ref, chi_ref):
    assert k <= 64
    n = S.shape[1]
    g = S
    while g.shape[1] > 64:
        half = g.shape[1] // 2
        g = jnp.maximum(g[:, :half], g[:, half:])
    lo0 = lax.bitcast_convert_type(jnp.min(g, axis=1, keepdims=True), jnp.int32)
    hi0 = lax.bitcast_convert_type(jnp.max(g, axis=1, keepdims=True), jnp.int32)
    lo_ref[...] = lo0
    hi_ref[...] = hi0
    clo_ref[...] = jnp.full(lo0.shape, float(n), jnp.float32)
    chi_ref[...] = jnp.zeros(lo0.shape, jnp.float32)

    def probe(lo, hi, clo, chi, mid):
        t = lax.bitcast_convert_type(mid, jnp.float32)
        cnt = jnp.sum((S >= t).astype(jnp.float32), axis=1, keepdims=True)
        ge = cnt >= float(k)
        eq = cnt == float(k)
        new_lo = jnp.where(ge, mid, lo)
        new_hi = jnp.where(eq, mid, jnp.where(ge, hi, mid - 1))
        new_clo = jnp.where(ge, cnt, clo)
        new_chi = jnp.where(ge, chi, cnt)
        return new_lo, new_hi, new_clo, new_chi

    def body(_):
        lo, hi = lo_ref[...], hi_ref[...]
        clo, chi = clo_ref[...], chi_ref[...]
        mid = lo + (hi - lo + 1) // 2
        lo, hi, clo, chi = probe(lo, hi, clo, chi, mid)
        span = (hi - lo + 1).astype(jnp.float32)
        off = (clo - float(k)) / (clo - chi) * span
        offi = jnp.clip(off.astype(jnp.int32), 1, jnp.maximum(hi - lo, 1))
        lo, hi, clo, chi = probe(lo, hi, clo, chi, lo + offi)
        lo_ref[...], hi_ref[...] = lo, hi
        clo_ref[...], chi_ref[...] = clo, chi
        return jnp.max(hi - lo) > 0

    lax.while_loop(lambda c: c, body, jnp.max(hi0 - lo0) > 0)
    t = lax.bitcast_convert_type(lo_ref[...], jnp.float32)
    return jnp.where(S >= t, S, 0.0)


def _embed_body(x_ref, w1_ref, b1_ref, w2_ref, b2_ref, h_ref):
    h = jax.nn.relu(_mm(x_ref[0], w1_ref[...]) + b1_ref[...])
    h_ref[0] = jax.nn.relu(_mm(h, w2_ref[...]) + b2_ref[...])


def _msgpass_body(hrows_ref, hfull_ref, wu_ref, wm_ref, out_ref,
                  lo_ref, hi_ref, clo_ref, chi_ref, *, k):
    hr = hrows_ref[0]
    hb = hfull_ref[0]
    A = _topk_mask(_gram(hr, hb), k, lo_ref, hi_ref, clo_ref, chi_ref)
    m = _mm(A, hb)
    out_ref[0] = jax.nn.relu(_mm(hr, wu_ref[...]) + _mm(m, wm_ref[...]))


def _final_body(hrows_ref, hfull_ref, out_ref,
                lo_ref, hi_ref, clo_ref, chi_ref, *, k):
    out_ref[0] = _topk_mask(_gram(hrows_ref[0], hfull_ref[0]), k,
                            lo_ref, hi_ref, clo_ref, chi_ref)


def kernel(x, W_emb1, b_emb1, W_emb2, b_emb2, W_msg, W_upd):
    bs, n, f = x.shape
    hid = W_emb1.shape[1]
    iters = W_msg.shape[0]
    k = 50
    r = min(1024, n)
    rb = n // r

    b1 = b_emb1.reshape(1, hid)
    b2 = b_emb2.reshape(1, hid)

    h = pl.pallas_call(
        _embed_body,
        grid=(bs,),
        in_specs=[
            pl.BlockSpec((1, n, f), lambda b: (b, 0, 0)),
            pl.BlockSpec((f, hid), lambda b: (0, 0)),
            pl.BlockSpec((1, hid), lambda b: (0, 0)),
            pl.BlockSpec((hid, hid), lambda b: (0, 0)),
            pl.BlockSpec((1, hid), lambda b: (0, 0)),
        ],
        out_specs=pl.BlockSpec((1, n, hid), lambda b: (b, 0, 0)),
        out_shape=jax.ShapeDtypeStruct((bs, n, hid), jnp.float32),
    )(x, W_emb1, b1, W_emb2, b2)

    for i in range(iters):
        h = pl.pallas_call(
            functools.partial(_msgpass_body, k=k),
            grid=(bs, rb),
            in_specs=[
                pl.BlockSpec((1, r, hid), lambda b, j: (b, j, 0)),
                pl.BlockSpec((1, n, hid), lambda b, j: (b, 0, 0)),
                pl.BlockSpec((hid, hid), lambda b, j: (0, 0)),
                pl.BlockSpec((hid, hid), lambda b, j: (0, 0)),
            ],
            out_specs=pl.BlockSpec((1, r, hid), lambda b, j: (b, j, 0)),
            out_shape=jax.ShapeDtypeStruct((bs, n, hid), jnp.float32),
            scratch_shapes=[pltpu.VMEM((r, 1), jnp.int32),
                            pltpu.VMEM((r, 1), jnp.int32),
                            pltpu.VMEM((r, 1), jnp.float32),
                            pltpu.VMEM((r, 1), jnp.float32)],
        )(h, h, W_upd[i], W_msg[i])

    return pl.pallas_call(
        functools.partial(_final_body, k=k),
        grid=(bs, rb),
        in_specs=[
            pl.BlockSpec((1, r, hid), lambda b, j: (b, j, 0)),
            pl.BlockSpec((1, n, hid), lambda b, j: (b, 0, 0)),
        ],
        out_specs=pl.BlockSpec((1, r, n), lambda b, j: (b, j, 0)),
        out_shape=jax.ShapeDtypeStruct((bs, n, n), jnp.float32),
        scratch_shapes=[pltpu.VMEM((r, 1), jnp.int32),
                        pltpu.VMEM((r, 1), jnp.int32),
                        pltpu.VMEM((r, 1), jnp.float32),
                        pltpu.VMEM((r, 1), jnp.float32)],
    )(h, h)
